# baseline (speedup 1.0000x reference)
import functools

import jax
import jax.numpy as jnp
from jax import lax
from jax.experimental import pallas as pl
from jax.experimental.pallas import tpu as pltpu
from jax.experimental.pallas import tpu_sc as plsc

_N = 10000
_E = 320000
_D = 128
_DH = _D // 2
_NC = 2
_NS = 16
_K = 128
_NCH = 160
_EP = _NCH * _K
_EPAD = _NS * _EP
_NP = 10240
_RP = _NP // _NS
_NBUF = 4
_BS = 1000
_NB = _N // _BS


def _sc_agg_body(xw0_hbm, xw1_hbm, row_hbm, col_hbm, sel_hbm, zrows_hbm,
                 out_hbm, row_v, col_v, tab_v, *rest):
    bufs = rest[:_NBUF]
    acc_sh = rest[_NBUF]
    gsems = rest[_NBUF + 1:2 * _NBUF + 1]
    ssems = rest[2 * _NBUF + 1:]
    c = lax.axis_index("c")
    s = lax.axis_index("s")
    pltpu.sync_copy(row_hbm.at[s], row_v)
    pltpu.sync_copy(col_hbm.at[s], col_v)
    pltpu.sync_copy(sel_hbm.at[c], tab_v)

    def translate(j, carry):
        for jj in range(_K // 16):
            cols16 = col_v[j, pl.ds(jj * 16, 16)]
            col_v[j, pl.ds(jj * 16, 16)] = plsc.load_gather(tab_v, [cols16])
        return carry

    lax.fori_loop(0, _NCH, translate, 0)

    H = _NBUF // 2
    rbase = s * _RP

    def g_start(j, b, xw_hbm):
        pltpu.async_copy(xw_hbm.at[col_v.at[j]], bufs[b], gsems[b])

    def g_wait(b, xw_hbm):
        pltpu.make_async_copy(xw_hbm.at[col_v.at[0]], bufs[b],
                              gsems[b]).wait()

    def s_start(j, b):
        pltpu.async_copy(bufs[b], acc_sh.at[row_v.at[j]], ssems[b], add=True)

    def s_wait(b):
        pltpu.make_async_copy(bufs[b], acc_sh.at[row_v.at[0]],
                              ssems[b]).wait()

    for half, xw_hbm in enumerate((xw0_hbm, xw1_hbm)):
        for b in range(H):
            g_start(b, b, xw_hbm)
        pltpu.sync_copy(zrows_hbm, acc_sh.at[pl.ds(rbase, _RP)])
        plsc.subcore_barrier()

        for b in range(_NBUF):
            g_wait(b, xw_hbm)
            s_start(b, b)
            s_wait(b)
            bp = (b + H) % _NBUF
            g_start(b + H, bp, xw_hbm)

        def octet(t, carry):
            j0 = t * _NBUF
            for b in range(_NBUF):
                j = j0 + b
                g_wait(b, xw_hbm)
                s_start(j, b)
                s_wait(b)
                bp = (b + H) % _NBUF
                g_start(j + H, bp, xw_hbm)
            return carry

        lax.fori_loop(1, _NCH // _NBUF - 1, octet, 0)

        for b in range(_NBUF):
            j = _NCH - _NBUF + b
            g_wait(b, xw_hbm)
            s_start(j, b)
            s_wait(b)
            if b < H:
                bp = (b + H) % _NBUF
                g_start(j + H, bp, xw_hbm)

        plsc.subcore_barrier()
        pltpu.sync_copy(acc_sh.at[pl.ds(rbase, _RP)],
                        out_hbm.at[c, half, pl.ds(rbase, _RP)])
        plsc.subcore_barrier()


@functools.lru_cache(maxsize=1)
def _sc_agg():
    mesh = plsc.VectorSubcoreMesh(core_axis_name="c", subcore_axis_name="s")
    return pl.kernel(
        _sc_agg_body,
        mesh=mesh,
        compiler_params=pltpu.CompilerParams(needs_layout_passes=False,
                                             use_tc_tiling_on_sc=False),
        out_type=jax.ShapeDtypeStruct((_NC, 2, _NP, _DH), jnp.float32),
        scratch_types=(
            [pltpu.VMEM((_NCH, _K), jnp.int32),
             pltpu.VMEM((_NCH, _K), jnp.int32),
             pltpu.VMEM((_N,), jnp.int32)]
            + [pltpu.VMEM((_K, _DH), jnp.float32)
               for _ in range(_NBUF)]
            + [pltpu.VMEM_SHARED((_NP, _DH), jnp.float32)]
            + [pltpu.SemaphoreType.DMA for _ in range(2 * _NBUF)]
        ),
    )


def _xw_body(x_ref, w_ref, o0_ref, o1_ref):
    xw = jnp.dot(x_ref[...], w_ref[...], preferred_element_type=jnp.float32)
    o0_ref[...] = xw[:, :_DH]
    o1_ref[...] = xw[:, _DH:]


def _csum_body(agg_ref, b_ref, pw_ref, csum_ref):
    z = jnp.concatenate([agg_ref[0, 0], agg_ref[0, 1]], axis=1) + b_ref[...]
    z = jnp.where(z > 0, z, pw_ref[...] * z)

    @pl.when(pl.program_id(0) == 0)
    def _init():
        csum_ref[...] = jnp.zeros_like(csum_ref)

    csum_ref[...] += jnp.sum(z, axis=0, keepdims=True)


def _loss_body(aggs_ref, b_ref, pw_ref, csum_ref, wbil_ref, bb_ref, out_ref):
    b = b_ref[...]
    pw = pw_ref[...]
    z = jnp.concatenate([aggs_ref[0, 0], aggs_ref[0, 1]], axis=1) + b
    z = jnp.where(z > 0, z, pw * z)
    zn = jnp.concatenate([aggs_ref[1, 0], aggs_ref[1, 1]], axis=1) + b
    zn = jnp.where(zn > 0, zn, pw * zn)
    m = csum_ref[...] * (1.0 / _N)
    g = 1.0 / (1.0 + jnp.exp(-m))
    gw = jnp.sum(wbil_ref[...] * g, axis=1, keepdims=True)
    bb = bb_ref[0, 0]
    sv = jnp.dot(z, gw, preferred_element_type=jnp.float32) + bb
    snv = jnp.dot(zn, gw, preferred_element_type=jnp.float32) + bb
    part = (jnp.sum(jnp.maximum(sv, 0.0) - sv
                    + jnp.log(1.0 + jnp.exp(-jnp.abs(sv))))
            + jnp.sum(jnp.maximum(snv, 0.0)
                      + jnp.log(1.0 + jnp.exp(-jnp.abs(snv)))))

    @pl.when(pl.program_id(0) == 0)
    def _init():
        out_ref[0, 0] = 0.0

    out_ref[0, 0] += part * (1.0 / (2.0 * _N))


def kernel(x, edges, perm, W_enc, b_enc, prelu_w, W_bil, b_bil):
    npad = _EPAD - _E
    padr = _N + (jnp.arange(npad, dtype=jnp.int32) % (_NP - _N))
    row3 = jnp.concatenate([edges[:, 0], padr]).reshape(_NS, _NCH, _K)
    padc = jnp.zeros((npad,), jnp.int32)
    col3 = jnp.concatenate([edges[:, 1], padc]).reshape(_NS, _NCH, _K)
    sel = jnp.stack([jnp.arange(_N, dtype=jnp.int32),
                     perm.astype(jnp.int32)])
    zrows = jnp.zeros((_RP, _DH), jnp.float32)

    xw0, xw1 = pl.pallas_call(
        _xw_body,
        grid=(_NB,),
        in_specs=[pl.BlockSpec((_BS, _D), lambda i: (i, 0)),
                  pl.BlockSpec((_D, _D), lambda i: (0, 0))],
        out_specs=[pl.BlockSpec((_BS, _DH), lambda i: (i, 0)),
                   pl.BlockSpec((_BS, _DH), lambda i: (i, 0))],
        out_shape=[jax.ShapeDtypeStruct((_N, _DH), jnp.float32),
                   jax.ShapeDtypeStruct((_N, _DH), jnp.float32)],
    )(x, W_enc)

    aggs = _sc_agg()(xw0, xw1, row3, col3, sel, zrows)

    b2 = b_enc.reshape(1, _D)
    pw2 = prelu_w.reshape(1, _D)
    csum = pl.pallas_call(
        _csum_body,
        grid=(_NB,),
        in_specs=[pl.BlockSpec((1, 2, _BS, _DH), lambda i: (0, 0, i, 0)),
                  pl.BlockSpec((1, _D), lambda i: (0, 0)),
                  pl.BlockSpec((1, _D), lambda i: (0, 0))],
        out_specs=pl.BlockSpec((1, _D), lambda i: (0, 0)),
        out_shape=jax.ShapeDtypeStruct((1, _D), jnp.float32),
    )(aggs, b2, pw2)

    loss2 = pl.pallas_call(
        _loss_body,
        grid=(_NB,),
        in_specs=[pl.BlockSpec((_NC, 2, _BS, _DH), lambda i: (0, 0, i, 0)),
                  pl.BlockSpec((1, _D), lambda i: (0, 0)),
                  pl.BlockSpec((1, _D), lambda i: (0, 0)),
                  pl.BlockSpec((1, _D), lambda i: (0, 0)),
                  pl.BlockSpec((_D, _D), lambda i: (0, 0)),
                  pl.BlockSpec(memory_space=pltpu.SMEM)],
        out_specs=pl.BlockSpec(memory_space=pltpu.SMEM),
        out_shape=jax.ShapeDtypeStruct((1, 1), jnp.float32),
    )(aggs, b2, pw2, csum, W_bil, b_bil.reshape(1, 1))

    return loss2[0, 0]


# speedup vs baseline: 2.6462x; 2.6462x over previous
import functools

import jax
import jax.numpy as jnp
from jax import lax
from jax.experimental import pallas as pl
from jax.experimental.pallas import tpu as pltpu
from jax.experimental.pallas import tpu_sc as plsc

_N = 10000
_E = 320000
_D = 128
_DH = _D // 2
_NC = 2
_NS = 16
_K = 80
_NCH = 250
_EP = _NCH * _K
_EPAD = _NS * _EP
_NP = 10240
_RP = _NP // _NS
_NBUF = 5
_BS = 1000
_NB = _N // _BS


def _sc_agg_body(xw0_hbm, xw1_hbm, row_hbm, col_hbm, sel_hbm, zrows_hbm,
                 out_hbm, row_v, col_v, tab_v, *rest):
    bufs = rest[:_NBUF]
    acc_sh = rest[_NBUF]
    gsems = rest[_NBUF + 1:2 * _NBUF + 1]
    ssems = rest[2 * _NBUF + 1:]
    c = lax.axis_index("c")
    s = lax.axis_index("s")
    pltpu.sync_copy(row_hbm.at[s], row_v)
    pltpu.sync_copy(col_hbm.at[s], col_v)
    pltpu.sync_copy(sel_hbm.at[c], tab_v)

    def translate(j, carry):
        for jj in range(_K // 16):
            cols16 = col_v[j, pl.ds(jj * 16, 16)]
            col_v[j, pl.ds(jj * 16, 16)] = plsc.load_gather(tab_v, [cols16])
        return carry

    lax.fori_loop(0, _NCH, translate, 0)

    rbase = s * _RP

    def g_start(j, b, xw_hbm):
        pltpu.async_copy(xw_hbm.at[col_v.at[j]], bufs[b], gsems[b])

    def g_wait(b, xw_hbm):
        pltpu.make_async_copy(xw_hbm.at[col_v.at[0]], bufs[b],
                              gsems[b]).wait()

    def s_start(j, b):
        pltpu.async_copy(bufs[b], acc_sh.at[row_v.at[j]], ssems[b], add=True)

    def s_wait(b):
        pltpu.make_async_copy(bufs[b], acc_sh.at[row_v.at[0]],
                              ssems[b]).wait()

    for half, xw_hbm in enumerate((xw0_hbm, xw1_hbm)):
        for b in range(_NBUF):
            g_start(b, b, xw_hbm)
        pltpu.sync_copy(zrows_hbm, acc_sh.at[pl.ds(rbase, _RP)])
        plsc.subcore_barrier()

        def turn(t, carry):
            j0 = t * _NBUF
            for b in range(_NBUF):
                j = j0 + b
                g_wait(b, xw_hbm)
                s_start(j, b)
                s_wait(b)
                g_start(j + _NBUF, b, xw_hbm)
            return carry

        lax.fori_loop(0, _NCH // _NBUF - 1, turn, 0)

        for b in range(_NBUF):
            j = _NCH - _NBUF + b
            g_wait(b, xw_hbm)
            s_start(j, b)
            s_wait(b)

        plsc.subcore_barrier()
        pltpu.sync_copy(acc_sh.at[pl.ds(rbase, _RP)],
                        out_hbm.at[c, half, pl.ds(rbase, _RP)])
        plsc.subcore_barrier()


@functools.lru_cache(maxsize=1)
def _sc_agg():
    mesh = plsc.VectorSubcoreMesh(core_axis_name="c", subcore_axis_name="s")
    return pl.kernel(
        _sc_agg_body,
        mesh=mesh,
        compiler_params=pltpu.CompilerParams(needs_layout_passes=False,
                                             use_tc_tiling_on_sc=False),
        out_type=jax.ShapeDtypeStruct((_NC, 2, _NP, _DH), jnp.float32),
        scratch_types=(
            [pltpu.VMEM((_NCH, _K), jnp.int32),
             pltpu.VMEM((_NCH, _K), jnp.int32),
             pltpu.VMEM((_N,), jnp.int32)]
            + [pltpu.VMEM((_K, _DH), jnp.float32)
               for _ in range(_NBUF)]
            + [pltpu.VMEM_SHARED((_NP, _DH), jnp.float32)]
            + [pltpu.SemaphoreType.DMA for _ in range(2 * _NBUF)]
        ),
    )


def _xw_body(x_ref, w_ref, o0_ref, o1_ref):
    xw = jnp.dot(x_ref[...], w_ref[...], preferred_element_type=jnp.float32)
    o0_ref[...] = xw[:, :_DH]
    o1_ref[...] = xw[:, _DH:]


def _csum_body(agg_ref, b_ref, pw_ref, csum_ref):
    z = jnp.concatenate([agg_ref[0, 0], agg_ref[0, 1]], axis=1) + b_ref[...]
    z = jnp.where(z > 0, z, pw_ref[...] * z)

    @pl.when(pl.program_id(0) == 0)
    def _init():
        csum_ref[...] = jnp.zeros_like(csum_ref)

    csum_ref[...] += jnp.sum(z, axis=0, keepdims=True)


def _loss_body(aggs_ref, b_ref, pw_ref, csum_ref, wbil_ref, bb_ref, out_ref):
    b = b_ref[...]
    pw = pw_ref[...]
    z = jnp.concatenate([aggs_ref[0, 0], aggs_ref[0, 1]], axis=1) + b
    z = jnp.where(z > 0, z, pw * z)
    zn = jnp.concatenate([aggs_ref[1, 0], aggs_ref[1, 1]], axis=1) + b
    zn = jnp.where(zn > 0, zn, pw * zn)
    m = csum_ref[...] * (1.0 / _N)
    g = 1.0 / (1.0 + jnp.exp(-m))
    gw = jnp.sum(wbil_ref[...] * g, axis=1, keepdims=True)
    bb = bb_ref[0, 0]
    sv = jnp.dot(z, gw, preferred_element_type=jnp.float32) + bb
    snv = jnp.dot(zn, gw, preferred_element_type=jnp.float32) + bb
    part = (jnp.sum(jnp.maximum(sv, 0.0) - sv
                    + jnp.log(1.0 + jnp.exp(-jnp.abs(sv))))
            + jnp.sum(jnp.maximum(snv, 0.0)
                      + jnp.log(1.0 + jnp.exp(-jnp.abs(snv)))))

    @pl.when(pl.program_id(0) == 0)
    def _init():
        out_ref[0, 0] = 0.0

    out_ref[0, 0] += part * (1.0 / (2.0 * _N))


def kernel(x, edges, perm, W_enc, b_enc, prelu_w, W_bil, b_bil):
    npad = _EPAD - _E
    padr = _N + (jnp.arange(npad, dtype=jnp.int32) % (_NP - _N))
    row3 = jnp.concatenate([edges[:, 0], padr]).reshape(_NS, _NCH, _K)
    padc = jnp.zeros((npad,), jnp.int32)
    col3 = jnp.concatenate([edges[:, 1], padc]).reshape(_NS, _NCH, _K)
    sel = jnp.stack([jnp.arange(_N, dtype=jnp.int32),
                     perm.astype(jnp.int32)])
    zrows = jnp.zeros((_RP, _DH), jnp.float32)

    xw0, xw1 = pl.pallas_call(
        _xw_body,
        grid=(_NB,),
        in_specs=[pl.BlockSpec((_BS, _D), lambda i: (i, 0)),
                  pl.BlockSpec((_D, _D), lambda i: (0, 0))],
        out_specs=[pl.BlockSpec((_BS, _DH), lambda i: (i, 0)),
                   pl.BlockSpec((_BS, _DH), lambda i: (i, 0))],
        out_shape=[jax.ShapeDtypeStruct((_N, _DH), jnp.float32),
                   jax.ShapeDtypeStruct((_N, _DH), jnp.float32)],
    )(x, W_enc)

    aggs = _sc_agg()(xw0, xw1, row3, col3, sel, zrows)

    b2 = b_enc.reshape(1, _D)
    pw2 = prelu_w.reshape(1, _D)
    csum = pl.pallas_call(
        _csum_body,
        grid=(_NB,),
        in_specs=[pl.BlockSpec((1, 2, _BS, _DH), lambda i: (0, 0, i, 0)),
                  pl.BlockSpec((1, _D), lambda i: (0, 0)),
                  pl.BlockSpec((1, _D), lambda i: (0, 0))],
        out_specs=pl.BlockSpec((1, _D), lambda i: (0, 0)),
        out_shape=jax.ShapeDtypeStruct((1, _D), jnp.float32),
    )(aggs, b2, pw2)

    loss2 = pl.pallas_call(
        _loss_body,
        grid=(_NB,),
        in_specs=[pl.BlockSpec((_NC, 2, _BS, _DH), lambda i: (0, 0, i, 0)),
                  pl.BlockSpec((1, _D), lambda i: (0, 0)),
                  pl.BlockSpec((1, _D), lambda i: (0, 0)),
                  pl.BlockSpec((1, _D), lambda i: (0, 0)),
                  pl.BlockSpec((_D, _D), lambda i: (0, 0)),
                  pl.BlockSpec(memory_space=pltpu.SMEM)],
        out_specs=pl.BlockSpec(memory_space=pltpu.SMEM),
        out_shape=jax.ShapeDtypeStruct((1, 1), jnp.float32),
    )(aggs, b2, pw2, csum, W_bil, b_bil.reshape(1, 1))

    return loss2[0, 0]


# speedup vs baseline: 3.7866x; 1.4310x over previous
import functools

import jax
import jax.numpy as jnp
from jax import lax
from jax.experimental import pallas as pl
from jax.experimental.pallas import tpu as pltpu
from jax.experimental.pallas import tpu_sc as plsc

_N = 10000
_E = 320000
_D = 128
_NC = 2
_NS = 16
_K = 80
_NCH = 250
_EP = _NCH * _K
_NP = 10240
_RP = _NP // _NS
_NBUF = 5
_BS = 1000
_NB = _N // _BS


def _sc_agg_body(xw_hbm, row_hbm, col_hbm, sel_hbm, zrows_hbm,
                 out_hbm, row_v, col_v, tab_v, *rest):
    bufs = rest[:_NBUF]
    acc_sh = rest[_NBUF]
    gsems = rest[_NBUF + 1:2 * _NBUF + 1]
    ssems = rest[2 * _NBUF + 1:]
    c = lax.axis_index("c")
    s = lax.axis_index("s")
    pltpu.sync_copy(row_hbm.at[s], row_v)
    pltpu.sync_copy(col_hbm.at[s], col_v)
    pltpu.sync_copy(sel_hbm.at[c], tab_v)

    def translate(j, carry):
        for jj in range(_K // 16):
            cols16 = col_v[j, pl.ds(jj * 16, 16)]
            col_v[j, pl.ds(jj * 16, 16)] = plsc.load_gather(tab_v, [cols16])
        return carry

    lax.fori_loop(0, _NCH, translate, 0)

    rbase = s * _RP

    def g_start(j, b):
        pltpu.async_copy(xw_hbm.at[col_v.at[j]], bufs[b], gsems[b])

    def g_wait(b):
        pltpu.make_async_copy(xw_hbm.at[col_v.at[0]], bufs[b],
                              gsems[b]).wait()

    def s_start(j, b):
        pltpu.async_copy(bufs[b], acc_sh.at[row_v.at[j]], ssems[b], add=True)

    def s_wait(b):
        pltpu.make_async_copy(bufs[b], acc_sh.at[row_v.at[0]],
                              ssems[b]).wait()

    for b in range(_NBUF):
        g_start(b, b)
    pltpu.sync_copy(zrows_hbm, acc_sh.at[pl.ds(rbase, _RP)])
    plsc.subcore_barrier()

    def turn(t, carry):
        j0 = t * _NBUF
        for b in range(_NBUF):
            j = j0 + b
            g_wait(b)
            s_start(j, b)
            s_wait(b)
            g_start(j + _NBUF, b)
        return carry

    lax.fori_loop(0, _NCH // _NBUF - 1, turn, 0)

    for b in range(_NBUF):
        j = _NCH - _NBUF + b
        g_wait(b)
        s_start(j, b)
        s_wait(b)

    plsc.subcore_barrier()
    pltpu.sync_copy(acc_sh.at[pl.ds(rbase, _RP)],
                    out_hbm.at[c, pl.ds(rbase, _RP)])


@functools.lru_cache(maxsize=1)
def _sc_agg():
    mesh = plsc.VectorSubcoreMesh(core_axis_name="c", subcore_axis_name="s")
    return pl.kernel(
        _sc_agg_body,
        mesh=mesh,
        compiler_params=pltpu.CompilerParams(needs_layout_passes=False,
                                             use_tc_tiling_on_sc=False),
        out_type=jax.ShapeDtypeStruct((_NC, _NP, _D), jnp.bfloat16),
        scratch_types=(
            [pltpu.VMEM((_NCH, _K), jnp.int32),
             pltpu.VMEM((_NCH, _K), jnp.int32),
             pltpu.VMEM((_N,), jnp.int32)]
            + [pltpu.VMEM((_K, _D), jnp.bfloat16)
               for _ in range(_NBUF)]
            + [pltpu.VMEM_SHARED((_NP, _D), jnp.bfloat16)]
            + [pltpu.SemaphoreType.DMA for _ in range(2 * _NBUF)]
        ),
    )


def _xw_body(x_ref, w_ref, o_ref):
    xw = jnp.dot(x_ref[...], w_ref[...], preferred_element_type=jnp.float32)
    o_ref[...] = xw.astype(jnp.bfloat16)


def _csum_body(agg_ref, b_ref, pw_ref, csum_ref):
    z = agg_ref[0].astype(jnp.float32) + b_ref[...]
    z = jnp.where(z > 0, z, pw_ref[...] * z)

    @pl.when(pl.program_id(0) == 0)
    def _init():
        csum_ref[...] = jnp.zeros_like(csum_ref)

    csum_ref[...] += jnp.sum(z, axis=0, keepdims=True)


def _loss_body(aggs_ref, b_ref, pw_ref, csum_ref, wbil_ref, bb_ref, out_ref):
    b = b_ref[...]
    pw = pw_ref[...]
    z = aggs_ref[0].astype(jnp.float32) + b
    z = jnp.where(z > 0, z, pw * z)
    zn = aggs_ref[1].astype(jnp.float32) + b
    zn = jnp.where(zn > 0, zn, pw * zn)
    m = csum_ref[...] * (1.0 / _N)
    g = 1.0 / (1.0 + jnp.exp(-m))
    gw = jnp.sum(wbil_ref[...] * g, axis=1, keepdims=True)
    bb = bb_ref[0, 0]
    sv = jnp.dot(z, gw, preferred_element_type=jnp.float32) + bb
    snv = jnp.dot(zn, gw, preferred_element_type=jnp.float32) + bb
    part = (jnp.sum(jnp.maximum(sv, 0.0) - sv
                    + jnp.log(1.0 + jnp.exp(-jnp.abs(sv))))
            + jnp.sum(jnp.maximum(snv, 0.0)
                      + jnp.log(1.0 + jnp.exp(-jnp.abs(snv)))))

    @pl.when(pl.program_id(0) == 0)
    def _init():
        out_ref[0, 0] = 0.0

    out_ref[0, 0] += part * (1.0 / (2.0 * _N))


def kernel(x, edges, perm, W_enc, b_enc, prelu_w, W_bil, b_bil):
    row3 = edges[:, 0].reshape(_NS, _NCH, _K)
    col3 = edges[:, 1].reshape(_NS, _NCH, _K)
    sel = jnp.stack([jnp.arange(_N, dtype=jnp.int32),
                     perm.astype(jnp.int32)])
    zrows = jnp.zeros((_RP, _D), jnp.bfloat16)

    xw = pl.pallas_call(
        _xw_body,
        grid=(_NB,),
        in_specs=[pl.BlockSpec((_BS, _D), lambda i: (i, 0)),
                  pl.BlockSpec((_D, _D), lambda i: (0, 0))],
        out_specs=pl.BlockSpec((_BS, _D), lambda i: (i, 0)),
        out_shape=jax.ShapeDtypeStruct((_N, _D), jnp.bfloat16),
    )(x, W_enc)

    aggs = _sc_agg()(xw, row3, col3, sel, zrows)

    b2 = b_enc.reshape(1, _D)
    pw2 = prelu_w.reshape(1, _D)
    csum = pl.pallas_call(
        _csum_body,
        grid=(_NB,),
        in_specs=[pl.BlockSpec((1, _BS, _D), lambda i: (0, i, 0)),
                  pl.BlockSpec((1, _D), lambda i: (0, 0)),
                  pl.BlockSpec((1, _D), lambda i: (0, 0))],
        out_specs=pl.BlockSpec((1, _D), lambda i: (0, 0)),
        out_shape=jax.ShapeDtypeStruct((1, _D), jnp.float32),
    )(aggs, b2, pw2)

    loss2 = pl.pallas_call(
        _loss_body,
        grid=(_NB,),
        in_specs=[pl.BlockSpec((_NC, _BS, _D), lambda i: (0, i, 0)),
                  pl.BlockSpec((1, _D), lambda i: (0, 0)),
                  pl.BlockSpec((1, _D), lambda i: (0, 0)),
                  pl.BlockSpec((1, _D), lambda i: (0, 0)),
                  pl.BlockSpec((_D, _D), lambda i: (0, 0)),
                  pl.BlockSpec(memory_space=pltpu.SMEM)],
        out_specs=pl.BlockSpec(memory_space=pltpu.SMEM),
        out_shape=jax.ShapeDtypeStruct((1, 1), jnp.float32),
    )(aggs, b2, pw2, csum, W_bil, b_bil.reshape(1, 1))

    return loss2[0, 0]


# speedup vs baseline: 3.7888x; 1.0006x over previous
import functools

import jax
import jax.numpy as jnp
from jax import lax
from jax.experimental import pallas as pl
from jax.experimental.pallas import tpu as pltpu
from jax.experimental.pallas import tpu_sc as plsc

_N = 10000
_E = 320000
_D = 128
_NC = 2
_NS = 16
_K = 80
_NCH = 250
_EP = _NCH * _K
_NP = 10240
_RP = _NP // _NS
_NBUF = 5
_BS = 1000
_NB = _N // _BS


def _sc_agg_body(xw_hbm, row_hbm, col_hbm, sel_hbm, zrows_hbm,
                 out_hbm, row_v, col_v, tab_v, *rest):
    bufs = rest[:_NBUF]
    acc_sh = rest[_NBUF]
    gsems = rest[_NBUF + 1:2 * _NBUF + 1]
    ssems = rest[2 * _NBUF + 1:]
    c = lax.axis_index("c")
    s = lax.axis_index("s")
    pltpu.sync_copy(row_hbm.at[s], row_v)
    pltpu.sync_copy(col_hbm.at[s], col_v)
    pltpu.sync_copy(sel_hbm.at[c], tab_v)

    def translate(j, carry):
        for jj in range(_K // 16):
            cols16 = col_v[j, pl.ds(jj * 16, 16)]
            col_v[j, pl.ds(jj * 16, 16)] = plsc.load_gather(tab_v, [cols16])
        return carry

    lax.fori_loop(0, _NCH, translate, 0)

    rbase = s * _RP

    def g_start(j, b):
        pltpu.async_copy(xw_hbm.at[col_v.at[j]], bufs[b], gsems[b])

    def g_wait(b):
        pltpu.make_async_copy(xw_hbm.at[col_v.at[0]], bufs[b],
                              gsems[b]).wait()

    def s_start(j, b):
        pltpu.async_copy(bufs[b], acc_sh.at[row_v.at[j]], ssems[b], add=True)

    def s_wait(b):
        pltpu.make_async_copy(bufs[b], acc_sh.at[row_v.at[0]],
                              ssems[b]).wait()

    for b in range(_NBUF):
        g_start(b, b)
    pltpu.sync_copy(zrows_hbm, acc_sh.at[pl.ds(rbase, _RP)])
    plsc.subcore_barrier()

    def turn(t, carry):
        j0 = t * _NBUF
        for b in range(_NBUF):
            j = j0 + b
            g_wait(b)
            s_start(j, b)
            s_wait(b)
            g_start(j + _NBUF, b)
        return carry

    lax.fori_loop(0, _NCH // _NBUF - 1, turn, 0)

    for b in range(_NBUF):
        j = _NCH - _NBUF + b
        g_wait(b)
        s_start(j, b)
        s_wait(b)

    plsc.subcore_barrier()
    pltpu.sync_copy(acc_sh.at[pl.ds(rbase, _RP)],
                    out_hbm.at[c, pl.ds(rbase, _RP)])


@functools.lru_cache(maxsize=1)
def _sc_agg():
    mesh = plsc.VectorSubcoreMesh(core_axis_name="c", subcore_axis_name="s")
    return pl.kernel(
        _sc_agg_body,
        mesh=mesh,
        compiler_params=pltpu.CompilerParams(needs_layout_passes=False,
                                             use_tc_tiling_on_sc=False),
        out_type=jax.ShapeDtypeStruct((_NC, _NP, _D), jnp.bfloat16),
        scratch_types=(
            [pltpu.VMEM((_NCH, _K), jnp.int32),
             pltpu.VMEM((_NCH, _K), jnp.int32),
             pltpu.VMEM((_N,), jnp.int32)]
            + [pltpu.VMEM((_K, _D), jnp.bfloat16)
               for _ in range(_NBUF)]
            + [pltpu.VMEM_SHARED((_NP, _D), jnp.bfloat16)]
            + [pltpu.SemaphoreType.DMA for _ in range(2 * _NBUF)]
        ),
    )


def _xw_body(x_ref, w_ref, o_ref):
    xw = jnp.dot(x_ref[...], w_ref[...], preferred_element_type=jnp.float32)
    o_ref[...] = xw.astype(jnp.bfloat16)


def _epi_body(aggs_ref, b_ref, pw_ref, wbil_ref, bb_ref, out_ref, csum_ref):
    ph = pl.program_id(0)
    b = b_ref[...]
    pw = pw_ref[...]

    @pl.when(ph == 0)
    def _csum_phase():
        z = aggs_ref[0].astype(jnp.float32) + b
        z = jnp.where(z > 0, z, pw * z)

        @pl.when(pl.program_id(1) == 0)
        def _init():
            csum_ref[...] = jnp.zeros_like(csum_ref)

        csum_ref[...] += jnp.sum(z, axis=0, keepdims=True)

    @pl.when(ph == 1)
    def _loss_phase():
        z = aggs_ref[0].astype(jnp.float32) + b
        z = jnp.where(z > 0, z, pw * z)
        zn = aggs_ref[1].astype(jnp.float32) + b
        zn = jnp.where(zn > 0, zn, pw * zn)
        m = csum_ref[...] * (1.0 / _N)
        g = 1.0 / (1.0 + jnp.exp(-m))
        gw = jnp.sum(wbil_ref[...] * g, axis=1, keepdims=True)
        bb = bb_ref[0, 0]
        sv = jnp.dot(z, gw, preferred_element_type=jnp.float32) + bb
        snv = jnp.dot(zn, gw, preferred_element_type=jnp.float32) + bb
        part = (jnp.sum(jnp.maximum(sv, 0.0) - sv
                        + jnp.log(1.0 + jnp.exp(-jnp.abs(sv))))
                + jnp.sum(jnp.maximum(snv, 0.0)
                          + jnp.log(1.0 + jnp.exp(-jnp.abs(snv)))))

        @pl.when(pl.program_id(1) == 0)
        def _init():
            out_ref[0, 0] = 0.0

        out_ref[0, 0] += part * (1.0 / (2.0 * _N))


def kernel(x, edges, perm, W_enc, b_enc, prelu_w, W_bil, b_bil):
    row3 = edges[:, 0].reshape(_NS, _NCH, _K)
    col3 = edges[:, 1].reshape(_NS, _NCH, _K)
    sel = jnp.stack([jnp.arange(_N, dtype=jnp.int32),
                     perm.astype(jnp.int32)])
    zrows = jnp.zeros((_RP, _D), jnp.bfloat16)

    xw = pl.pallas_call(
        _xw_body,
        grid=(_NB,),
        in_specs=[pl.BlockSpec((_BS, _D), lambda i: (i, 0)),
                  pl.BlockSpec((_D, _D), lambda i: (0, 0))],
        out_specs=pl.BlockSpec((_BS, _D), lambda i: (i, 0)),
        out_shape=jax.ShapeDtypeStruct((_N, _D), jnp.bfloat16),
    )(x, W_enc)

    aggs = _sc_agg()(xw, row3, col3, sel, zrows)

    b2 = b_enc.reshape(1, _D)
    pw2 = prelu_w.reshape(1, _D)
    loss2 = pl.pallas_call(
        _epi_body,
        grid=(2, _NB),
        in_specs=[pl.BlockSpec((_NC, _BS, _D), lambda p, i: (0, i, 0)),
                  pl.BlockSpec((1, _D), lambda p, i: (0, 0)),
                  pl.BlockSpec((1, _D), lambda p, i: (0, 0)),
                  pl.BlockSpec((_D, _D), lambda p, i: (0, 0)),
                  pl.BlockSpec(memory_space=pltpu.SMEM)],
        out_specs=pl.BlockSpec(memory_space=pltpu.SMEM),
        out_shape=jax.ShapeDtypeStruct((1, 1), jnp.float32),
        scratch_shapes=[pltpu.VMEM((1, _D), jnp.float32)],
    )(aggs, b2, pw2, W_bil, b_bil.reshape(1, 1))

    return loss2[0, 0]
